# baseline (speedup 1.0000x reference)
import jax
import jax.numpy as jnp
from jax import lax
from jax.experimental import pallas as pl
from jax.experimental.pallas import tpu as pltpu
from jax.experimental.pallas import tpu_sc as plsc

BATCH = 16384
L = 16
NW = 4
CHUNK = BATCH // NW


def _sc_body(tau_hbm, inputs_hbm, theta_hbm, mu_hbm, out_hbm,
             idx_v, th_v, mu_v, tau_v, out_v, sem_g, sem_t):
    base = lax.axis_index("s") * CHUNK
    in_cp = pltpu.make_async_copy(inputs_hbm, idx_v.at[pl.ds(0, 1)], sem_g)
    in_cp.start()
    tau_cp = pltpu.make_async_copy(tau_hbm.at[pl.ds(base, CHUNK)], tau_v, sem_t)
    tau_cp.start()
    in_cp.wait()
    th_cp = pltpu.make_async_copy(theta_hbm.at[idx_v.at[pl.ds(0, 1)]], th_v.at[pl.ds(0, 1)], sem_g)
    mu_cp = pltpu.make_async_copy(mu_hbm.at[idx_v.at[pl.ds(0, 1)]], mu_v.at[pl.ds(0, 1)], sem_g)
    th_cp.start()
    mu_cp.start()
    th_cp.wait()
    mu_cp.wait()
    nth = -th_v[...][0]
    nm = -mu_v[...][0]
    tau_cp.wait()

    @plsc.parallel_loop(0, CHUNK, step=L, unroll=32)
    def _compute(i):
        x = tau_v[pl.ds(i, L)]
        out_v[pl.ds(i, L)] = 1.0 / (1.0 + jnp.exp(nth + nm * x))

    pltpu.sync_copy(out_v, out_hbm.at[pl.ds(base, CHUNK)])


@jax.jit
def _embed_sigmoid(tau, inputs, theta, mu):
    k = pl.kernel(
        _sc_body,
        out_type=jax.ShapeDtypeStruct((BATCH,), jnp.float32),
        mesh=plsc.VectorSubcoreMesh(core_axis_name="c", subcore_axis_name="s",
                                    num_cores=1, num_subcores=NW),
        scratch_types=[
            pltpu.VMEM((L,), jnp.int32),
            pltpu.VMEM((L,), jnp.float32),
            pltpu.VMEM((L,), jnp.float32),
            pltpu.VMEM((CHUNK,), jnp.float32),
            pltpu.VMEM((CHUNK,), jnp.float32),
            pltpu.SemaphoreType.DMA,
            pltpu.SemaphoreType.DMA,
        ],
    )
    return k(tau, inputs, theta, mu)


def kernel(tau, inputs, theta, mu):
    return _embed_sigmoid(tau, inputs, theta, mu)


# speedup vs baseline: 1.0147x; 1.0147x over previous
import jax
import jax.numpy as jnp
from jax import lax
from jax.experimental import pallas as pl
from jax.experimental.pallas import tpu as pltpu
from jax.experimental.pallas import tpu_sc as plsc

BATCH = 16384
L = 16
NW = 4
CHUNK = BATCH // NW


def _sc_body(tau_hbm, inputs_hbm, theta_hbm, mu_hbm, out_hbm,
             idx_v, th_v, mu_v, tau_v, out_v, sem_g, sem_t):
    base = lax.axis_index("s") * CHUNK
    in_cp = pltpu.make_async_copy(inputs_hbm, idx_v.at[pl.ds(0, 1)], sem_g)
    in_cp.start()
    tau_cp = pltpu.make_async_copy(tau_hbm.at[pl.ds(base, CHUNK)], tau_v, sem_t)
    tau_cp.start()
    in_cp.wait()
    th_cp = pltpu.make_async_copy(theta_hbm.at[idx_v.at[pl.ds(0, 1)]], th_v.at[pl.ds(0, 1)], sem_g)
    mu_cp = pltpu.make_async_copy(mu_hbm.at[idx_v.at[pl.ds(0, 1)]], mu_v.at[pl.ds(0, 1)], sem_g)
    th_cp.start()
    mu_cp.start()
    th_cp.wait()
    mu_cp.wait()
    nth = -th_v[...][0]
    nm = -mu_v[...][0]
    tau_cp.wait()

    @plsc.parallel_loop(0, CHUNK, step=L, unroll=16)
    def _compute(i):
        x = tau_v[pl.ds(i, L)]
        out_v[pl.ds(i, L)] = 1.0 / (1.0 + jnp.exp(nth + nm * x))

    pltpu.sync_copy(out_v, out_hbm.at[pl.ds(base, CHUNK)])


@jax.jit
def _embed_sigmoid(tau, inputs, theta, mu):
    k = pl.kernel(
        _sc_body,
        out_type=jax.ShapeDtypeStruct((BATCH,), jnp.float32),
        mesh=plsc.VectorSubcoreMesh(core_axis_name="c", subcore_axis_name="s",
                                    num_cores=1, num_subcores=NW),
        scratch_types=[
            pltpu.VMEM((L,), jnp.int32),
            pltpu.VMEM((L,), jnp.float32),
            pltpu.VMEM((L,), jnp.float32),
            pltpu.VMEM((CHUNK,), jnp.float32),
            pltpu.VMEM((CHUNK,), jnp.float32),
            pltpu.SemaphoreType.DMA,
            pltpu.SemaphoreType.DMA,
        ],
    )
    return k(tau, inputs, theta, mu)


def kernel(tau, inputs, theta, mu):
    return _embed_sigmoid(tau, inputs, theta, mu)
